# Initial kernel scaffold; baseline (speedup 1.0000x reference)
import jax
import jax.numpy as jnp
from jax.experimental import pallas as pl


def kernel(weight, mask, input):
    raise NotImplementedError("write your pallas kernel here")


# speedup vs baseline: 1.1000x; 1.1000x over previous
import functools

import jax
import jax.numpy as jnp
from jax import lax
from jax.experimental import pallas as pl
from jax.experimental.pallas import tpu as pltpu
from jax.experimental.pallas import tpu_sc as plsc

_NC = 2
_NS = 16
_NW = _NC * _NS

_D = 64
_CHUNK = 128


@functools.cache
def _make_gather(B):
    bpw = B // _NW
    nchunk = bpw // _CHUNK
    mesh = plsc.VectorSubcoreMesh(core_axis_name="c", subcore_axis_name="s")

    @functools.partial(
        pl.kernel,
        mesh=mesh,
        out_type=jax.ShapeDtypeStruct((B, _D), jnp.float32),
        scratch_types=[
            pltpu.VMEM((nchunk, _CHUNK), jnp.int32),
            pltpu.VMEM((_CHUNK, _D), jnp.float32),
            pltpu.SemaphoreType.DMA,
        ],
        compiler_params=pltpu.CompilerParams(use_tc_tiling_on_sc=False),
    )
    def gather_kernel(idx_hbm, table_hbm, out_hbm, idx_v, rows_v, gsem):
        wid = lax.axis_index("s") * _NC + lax.axis_index("c")
        base = wid * bpw
        pltpu.sync_copy(idx_hbm.at[wid], idx_v)

        def body(j, carry):
            pltpu.async_copy(table_hbm.at[idx_v.at[j]], rows_v, gsem).wait()
            pltpu.sync_copy(rows_v, out_hbm.at[pl.ds(base + j * _CHUNK, _CHUNK)])
            return carry

        lax.fori_loop(0, nchunk, body, 0)

    return gather_kernel


def kernel(weight, mask, input):
    b, f = input.shape
    B = b * f
    idx = input.reshape(_NW, B // _NW // _CHUNK, _CHUNK).astype(jnp.int32)
    out = _make_gather(B)(idx, weight)
    return out.reshape(b, f, _D)


# speedup vs baseline: 1.2263x; 1.1148x over previous
import functools

import jax
import jax.numpy as jnp
from jax import lax
from jax.experimental import pallas as pl
from jax.experimental.pallas import tpu as pltpu
from jax.experimental.pallas import tpu_sc as plsc

_NC = 2
_NS = 16
_NW = _NC * _NS

_D = 64
_G = 16


@functools.cache
def _make_gather(batch, fields):
    bpw = batch // _NW
    ngrp = bpw // _G
    mesh = plsc.VectorSubcoreMesh(core_axis_name="c", subcore_axis_name="s")

    @functools.partial(
        pl.kernel,
        mesh=mesh,
        out_type=jax.ShapeDtypeStruct((batch, fields, _D), jnp.float32),
        scratch_types=[
            pltpu.VMEM((bpw, fields), jnp.int32),
            pltpu.VMEM((2, _G, fields, _D), jnp.float32),
            pltpu.SemaphoreType.DMA,
            pltpu.SemaphoreType.DMA,
        ],
        compiler_params=pltpu.CompilerParams(use_tc_tiling_on_sc=False),
    )
    def gather_kernel(idx_hbm, table_hbm, out_hbm, idx_v, rows_v, gsem, wsem):
        wid = lax.axis_index("s") * _NC + lax.axis_index("c")
        base = wid * bpw
        pltpu.sync_copy(idx_hbm.at[pl.ds(base, bpw)], idx_v)

        def fire_group_gathers(g, buf):
            for k in range(_G):
                pltpu.async_copy(
                    table_hbm.at[idx_v.at[g * _G + k]], rows_v.at[buf, k],
                    gsem)

        def wait_group_gathers():
            for _ in range(_G):
                pltpu.make_async_copy(
                    table_hbm.at[idx_v.at[0]], rows_v.at[0, 0], gsem).wait()

        def fire_write(g, buf):
            pltpu.async_copy(
                rows_v.at[buf], out_hbm.at[pl.ds(base + g * _G, _G)], wsem)

        def wait_write():
            pltpu.make_async_copy(
                rows_v.at[0], out_hbm.at[pl.ds(base, _G)], wsem).wait()

        fire_group_gathers(0, 0)

        def body(g, carry):
            nxt = lax.rem(g + 1, 2)

            @pl.when(g + 1 < ngrp)
            def _prefetch():
                @pl.when(g >= 1)
                def _free_buf():
                    wait_write()

                fire_group_gathers(g + 1, nxt)

            wait_group_gathers()
            fire_write(g, lax.rem(g, 2))
            return carry

        lax.fori_loop(0, ngrp, body, 0)
        wait_write()
        wait_write()

    return gather_kernel


def kernel(weight, mask, input):
    b, f = input.shape
    return _make_gather(b, f)(input.astype(jnp.int32), weight)
